# baseline (speedup 1.0000x reference)
import functools

import jax
import jax.numpy as jnp
from jax import lax
from jax.experimental import pallas as pl
from jax.experimental.pallas import tpu as pltpu
from jax.experimental.pallas import tpu_sc as plsc

N_TOT = 8388608
NC = 2
NS = 16
L = 16
NW = NC * NS

S_SC = 5242880
NE = S_SC // NW
CH = 16384
NCH = NE // CH
U = 8
STEPS = CH // (U * L)

W = 512
ROWS = N_TOT // W
RB = 512
TCR0 = S_SC // W
NT = (ROWS - TCR0) // RB

_mesh = plsc.VectorSubcoreMesh(core_axis_name="c", subcore_axis_name="s")


@functools.partial(
    pl.kernel,
    mesh=_mesh,
    out_type=jax.ShapeDtypeStruct((NW, 3, L), jnp.int32),
    scratch_types=[
        pltpu.VMEM((CH,), jnp.int32),
        pltpu.VMEM((CH,), jnp.int32),
        pltpu.VMEM((CH,), jnp.int32),
        pltpu.VMEM((CH,), jnp.int32),
        pltpu.VMEM((3, L), jnp.int32),
        pltpu.SemaphoreType.DMA,
        pltpu.SemaphoreType.DMA,
    ],
)
def _sc_partial_counts(p_hbm, g_hbm, out_hbm, pb0, pb1, gb0, gb1, accv, sem0, sem1):
    wid = lax.axis_index("s") * NC + lax.axis_index("c")
    base = wid * NE
    pbufs = (pb0, pb1)
    gbufs = (gb0, gb1)
    sems = (sem0, sem1)

    def start(c):
        b = c % 2
        off = base + c * CH
        hp = pltpu.async_copy(p_hbm.at[pl.ds(off, CH)], pbufs[b], sems[b])
        hg = pltpu.async_copy(g_hbm.at[pl.ds(off, CH)], gbufs[b], sems[b])
        return hp, hg

    inflight = {0: start(0)}
    zero = jnp.zeros((L,), jnp.int32)
    acc_p, acc_g, acc_pg = zero, zero, zero

    for c in range(NCH):
        if c + 1 < NCH:
            inflight[c + 1] = start(c + 1)
        hp, hg = inflight.pop(c)
        hp.wait()
        hg.wait()
        b = c % 2
        pb = pbufs[b]
        gb = gbufs[b]

        def body(i, carry, pb=pb, gb=gb):
            ap, ag, apg = carry
            o0 = i * (U * L)
            for u in range(U):
                pv = pb[pl.ds(o0 + u * L, L)]
                gv = gb[pl.ds(o0 + u * L, L)]
                ap = ap + pv
                ag = ag + gv
                apg = apg + (pv & gv)
            return ap, ag, apg

        acc_p, acc_g, acc_pg = lax.fori_loop(
            0, STEPS, body, (acc_p, acc_g, acc_pg)
        )

    accv[0] = acc_p
    accv[1] = acc_g
    accv[2] = acc_pg
    pltpu.sync_copy(accv, out_hbm.at[wid])


def _tc_reduce_body(p_ref, g_ref, out_ref):
    i = pl.program_id(0)
    pv = p_ref[...]
    gv = g_ref[...]
    sp = jnp.sum(pv, axis=0, keepdims=True)
    sg = jnp.sum(gv, axis=0, keepdims=True)
    spg = jnp.sum(pv & gv, axis=0, keepdims=True)
    part = jnp.concatenate([sp, sg, spg], axis=0)

    @pl.when(i == 0)
    def _():
        out_ref[...] = part

    @pl.when(i > 0)
    def _():
        out_ref[...] = out_ref[...] + part


_tc_reduce = pl.pallas_call(
    _tc_reduce_body,
    grid=(NT,),
    in_specs=[
        pl.BlockSpec((RB, W), lambda i: (TCR0 // RB + i, 0)),
        pl.BlockSpec((RB, W), lambda i: (TCR0 // RB + i, 0)),
    ],
    out_specs=pl.BlockSpec((3, W), lambda i: (0, 0)),
    out_shape=jax.ShapeDtypeStruct((3, W), jnp.int32),
)


def _combine_body(sc_ref, tc_ref, conf_ref, out_ref):
    xs = sc_ref[...]
    s2 = jnp.sum(xs, axis=0)
    ssc = jnp.sum(s2, axis=1)
    stc = jnp.sum(tc_ref[...], axis=1)
    s = ssc + stc
    spf = s[0].astype(jnp.float32)
    sgf = s[1].astype(jnp.float32)
    spgf = s[2].astype(jnp.float32)
    c00 = jnp.float32(N_TOT) - spf - sgf + spgf
    c01 = sgf - spgf
    c10 = spf - spgf
    c11 = spgf
    ii = lax.broadcasted_iota(jnp.int32, (2, 2), 0)
    jj = lax.broadcasted_iota(jnp.int32, (2, 2), 1)
    cm = jnp.where(
        (ii == 0) & (jj == 0),
        c00,
        jnp.where((ii == 0) & (jj == 1), c01, jnp.where(jj == 0, c10, c11)),
    )
    out_ref[...] = conf_ref[...] + cm


_combine = pl.pallas_call(
    _combine_body,
    out_shape=jax.ShapeDtypeStruct((2, 2), jnp.float32),
)


def kernel(pred_labels, gt_labels, conf_matrix):
    sc_partials = _sc_partial_counts(pred_labels, gt_labels)
    p2d = pred_labels.reshape(ROWS, W)
    g2d = gt_labels.reshape(ROWS, W)
    tc_partials = _tc_reduce(p2d, g2d)
    return _combine(sc_partials, tc_partials, conf_matrix)


# speedup vs baseline: 2.1853x; 2.1853x over previous
import functools

import jax
import jax.numpy as jnp
from jax import lax
from jax.experimental import pallas as pl
from jax.experimental.pallas import tpu as pltpu
from jax.experimental.pallas import tpu_sc as plsc

N_TOT = 8388608
NC = 2
NS = 16
L = 16
NW = NC * NS
NE = N_TOT // NW
CH = 16384
NCH = NE // CH
NBUF = 3
U = 16
STEPS = CH // (U * L)

_mesh = plsc.VectorSubcoreMesh(core_axis_name="c", subcore_axis_name="s")


@functools.partial(
    pl.kernel,
    mesh=_mesh,
    out_type=jax.ShapeDtypeStruct((NW, 3, L), jnp.int32),
    scratch_types=(
        [pltpu.VMEM((CH,), jnp.int32) for _ in range(2 * NBUF)]
        + [pltpu.VMEM((3, L), jnp.int32)]
        + [pltpu.SemaphoreType.DMA for _ in range(NBUF)]
    ),
)
def _sc_partial_counts(p_hbm, g_hbm, out_hbm, *scr):
    pbufs = scr[0:NBUF]
    gbufs = scr[NBUF:2 * NBUF]
    accv = scr[2 * NBUF]
    sems = scr[2 * NBUF + 1:]

    wid = lax.axis_index("s") * NC + lax.axis_index("c")
    base = wid * NE

    def start(c):
        b = c % NBUF
        off = base + c * CH
        hp = pltpu.async_copy(p_hbm.at[pl.ds(off, CH)], pbufs[b], sems[b])
        hg = pltpu.async_copy(g_hbm.at[pl.ds(off, CH)], gbufs[b], sems[b])
        return hp, hg

    inflight = {c: start(c) for c in range(NBUF - 1)}
    zero = jnp.zeros((L,), jnp.int32)
    acc_p, acc_g, acc_pg = zero, zero, zero

    for c in range(NCH):
        if c + NBUF - 1 < NCH:
            inflight[c + NBUF - 1] = start(c + NBUF - 1)
        hp, hg = inflight.pop(c)
        hp.wait()
        hg.wait()
        b = c % NBUF
        pb = pbufs[b]
        gb = gbufs[b]

        def body(i, carry, pb=pb, gb=gb):
            ap, ag, apg = carry
            o0 = i * (U * L)
            for u in range(U):
                pv = pb[pl.ds(o0 + u * L, L)]
                gv = gb[pl.ds(o0 + u * L, L)]
                ap = ap + pv
                ag = ag + gv
                apg = apg + (pv & gv)
            return ap, ag, apg

        acc_p, acc_g, acc_pg = lax.fori_loop(
            0, STEPS, body, (acc_p, acc_g, acc_pg)
        )

    accv[0] = acc_p
    accv[1] = acc_g
    accv[2] = acc_pg
    pltpu.sync_copy(accv, out_hbm.at[wid])


def _combine_body(part_ref, conf_ref, out_ref):
    x = part_ref[...]
    s2 = jnp.sum(x, axis=0)
    s = jnp.sum(s2, axis=1)
    spf = s[0].astype(jnp.float32)
    sgf = s[1].astype(jnp.float32)
    spgf = s[2].astype(jnp.float32)
    c00 = jnp.float32(N_TOT) - spf - sgf + spgf
    c01 = sgf - spgf
    c10 = spf - spgf
    c11 = spgf
    ii = lax.broadcasted_iota(jnp.int32, (2, 2), 0)
    jj = lax.broadcasted_iota(jnp.int32, (2, 2), 1)
    cm = jnp.where(
        (ii == 0) & (jj == 0),
        c00,
        jnp.where((ii == 0) & (jj == 1), c01, jnp.where(jj == 0, c10, c11)),
    )
    out_ref[...] = conf_ref[...] + cm


_combine = pl.pallas_call(
    _combine_body,
    out_shape=jax.ShapeDtypeStruct((2, 2), jnp.float32),
)


def kernel(pred_labels, gt_labels, conf_matrix):
    partials = _sc_partial_counts(pred_labels, gt_labels)
    return _combine(partials, conf_matrix)
